# baseline (speedup 1.0000x reference)
import functools

import jax
import jax.numpy as jnp
from jax import lax
from jax.experimental import pallas as pl
from jax.experimental.pallas import tpu as pltpu
from jax.experimental.pallas import tpu_sc as plsc

NC, NS = 2, 16
NW = NC * NS
LANES = 16
CHUNK = 16
GBUF = 5


@functools.cache
def _build(nb, seq, vocab, d):
    ppw = seq // NW
    ncp = ppw // CHUNK
    nun = ncp * nb
    vecs = d // LANES
    assert vecs & (vecs - 1) == 0
    shift = vecs.bit_length() - 1
    mesh = plsc.VectorSubcoreMesh(core_axis_name="c", subcore_axis_name="s")

    @functools.partial(
        pl.kernel,
        mesh=mesh,
        out_type=jax.ShapeDtypeStruct((nb * seq, d), jnp.float32),
        scratch_types=[
            pltpu.VMEM((ncp, nb, CHUNK), jnp.int32),
            *[pltpu.VMEM((CHUNK, d), jnp.float32) for _ in range(GBUF + 2)],
            *[pltpu.SemaphoreType.DMA for _ in range(2 * GBUF + 3)],
        ],
    )
    def emb(x_hbm, table_hbm, pe_hbm, out_hbm, idx_v, *bufs_sems):
        gbuf = list(bufs_sems[:GBUF])
        pbuf = list(bufs_sems[GBUF:GBUF + 2])
        sems = bufs_sems[GBUF + 2:]
        sg = list(sems[:GBUF])
        ss = list(sems[GBUF:2 * GBUF])
        sp = list(sems[2 * GBUF:2 * GBUF + 2])
        si = sems[2 * GBUF + 2]
        wid = lax.axis_index("s") * NC + lax.axis_index("c")
        pbase = wid * ppw
        gd = [None] * nun
        pd = [None] * ncp
        sd = [None] * nun

        cbase = pl.multiple_of(pbase // CHUNK, 8)
        idx_copies = [
            pltpu.async_copy(x_hbm.at[b, pl.ds(cbase, ncp)],
                             idx_v.at[:, b], si)
            for b in range(nb)
        ]

        def load_pe(c):
            pd[c] = pltpu.async_copy(
                pe_hbm.at[pl.ds(pbase + c * CHUNK, CHUNK)],
                pbuf[c % 2], sp[c % 2])

        def start(u):
            c, b = divmod(u, nb)
            gd[u] = pltpu.async_copy(
                table_hbm.at[idx_v.at[c, b]],
                gbuf[u % GBUF], sg[u % GBUF])

        def finish(u):
            c, b = divmod(u, nb)
            gd[u].wait()
            if b == 0:
                pd[c].wait()
            g, p = gbuf[u % GBUF], pbuf[c % 2]

            @plsc.parallel_loop(0, CHUNK * vecs, unroll=8)
            def vec(i):
                r = jax.lax.shift_right_logical(i, shift)
                col = (i & (vecs - 1)) * LANES
                plsc.addupdate(g.at[r, pl.ds(col, LANES)],
                               p[r, pl.ds(col, LANES)])

            sd[u] = pltpu.async_copy(
                g, out_hbm.at[pl.ds(b * seq + pbase + c * CHUNK, CHUNK)],
                ss[u % GBUF])
            if b == nb - 1 and c + 2 < ncp:
                load_pe(c + 2)

        load_pe(0)
        if ncp > 1:
            load_pe(1)
        for dcopy in idx_copies:
            dcopy.wait()
        lag = GBUF - 1
        for u in range(nun):
            if u >= GBUF:
                sd[u - GBUF].wait()
            start(u)
            if u >= lag:
                finish(u - lag)
        for u in range(nun - lag, nun):
            finish(u)
        for u in range(nun - GBUF, nun):
            sd[u].wait()

    return emb


def kernel(x, table, pe):
    b, s = x.shape
    vocab, d = table.shape
    x3 = x.astype(jnp.int32).reshape(b, s // CHUNK, CHUNK)
    out = _build(b, s, vocab, d)(x3, table, pe)
    return out.reshape(b, s, d)


# speedup vs baseline: 1.0059x; 1.0059x over previous
import functools

import jax
import jax.numpy as jnp
from jax import lax
from jax.experimental import pallas as pl
from jax.experimental.pallas import tpu as pltpu
from jax.experimental.pallas import tpu_sc as plsc

NC, NS = 2, 16
NW = NC * NS
LANES = 16
CHUNK = 16
GBUF = 5


@functools.cache
def _build(nb, seq, vocab, d):
    ppw = seq // NW
    ncp = ppw // CHUNK
    nun = ncp * nb
    vecs = d // LANES
    assert vecs & (vecs - 1) == 0
    shift = vecs.bit_length() - 1
    mesh = plsc.VectorSubcoreMesh(core_axis_name="c", subcore_axis_name="s")

    @functools.partial(
        pl.kernel,
        mesh=mesh,
        out_type=jax.ShapeDtypeStruct((nb, seq, d), jnp.float32),
        scratch_types=[
            pltpu.VMEM((nb, ppw), jnp.int32),
            *[pltpu.VMEM((CHUNK, d), jnp.float32) for _ in range(GBUF + 2)],
            *[pltpu.SemaphoreType.DMA for _ in range(2 * GBUF + 3)],
        ],
    )
    def emb(x_hbm, table_hbm, pe_hbm, out_hbm, idx_v, *bufs_sems):
        gbuf = list(bufs_sems[:GBUF])
        pbuf = list(bufs_sems[GBUF:GBUF + 2])
        sems = bufs_sems[GBUF + 2:]
        sg = list(sems[:GBUF])
        ss = list(sems[GBUF:2 * GBUF])
        sp = list(sems[2 * GBUF:2 * GBUF + 2])
        si = sems[2 * GBUF + 2]
        wid = lax.axis_index("s") * NC + lax.axis_index("c")
        pbase = wid * ppw
        gd = [None] * nun
        pd = [None] * ncp
        sd = [None] * nun

        idx_copy = pltpu.async_copy(
            x_hbm.at[:, pl.ds(pbase, ppw)], idx_v, si)

        def load_pe(c):
            pd[c] = pltpu.async_copy(
                pe_hbm.at[pl.ds(pbase + c * CHUNK, CHUNK)],
                pbuf[c % 2], sp[c % 2])

        def start(u):
            c, b = divmod(u, nb)
            gd[u] = pltpu.async_copy(
                table_hbm.at[idx_v.at[b, pl.ds(c * CHUNK, CHUNK)]],
                gbuf[u % GBUF], sg[u % GBUF])

        def finish(u):
            c, b = divmod(u, nb)
            gd[u].wait()
            if b == 0:
                pd[c].wait()
            g, p = gbuf[u % GBUF], pbuf[c % 2]

            @plsc.parallel_loop(0, CHUNK * vecs, unroll=8)
            def vec(i):
                r = jax.lax.shift_right_logical(i, shift)
                col = (i & (vecs - 1)) * LANES
                plsc.addupdate(g.at[r, pl.ds(col, LANES)],
                               p[r, pl.ds(col, LANES)])

            sd[u] = pltpu.async_copy(
                g, out_hbm.at[b, pl.ds(pbase + c * CHUNK, CHUNK)],
                ss[u % GBUF])
            if b == nb - 1 and c + 2 < ncp:
                load_pe(c + 2)

        load_pe(0)
        if ncp > 1:
            load_pe(1)
        idx_copy.wait()
        lag = GBUF - 1
        for u in range(nun):
            if u >= GBUF:
                sd[u - GBUF].wait()
            start(u)
            if u >= lag:
                finish(u - lag)
        for u in range(nun - lag, nun):
            finish(u)
        for u in range(nun - GBUF, nun):
            sd[u].wait()

    return emb


def kernel(x, table, pe):
    b, s = x.shape
    vocab, d = table.shape
    return _build(b, s, vocab, d)(x.astype(jnp.int32), table, pe)


# speedup vs baseline: 1.0450x; 1.0389x over previous
import functools

import jax
import jax.numpy as jnp
from jax import lax
from jax.experimental import pallas as pl
from jax.experimental.pallas import tpu as pltpu
from jax.experimental.pallas import tpu_sc as plsc

NC, NS = 2, 16
NW = NC * NS
LANES = 16
CHUNK = 16


@functools.cache
def _build(nb, seq, vocab, d):
    ppw = seq // NW
    ncp = ppw // CHUNK
    vecs = d // LANES
    assert vecs & (vecs - 1) == 0
    assert nb == 4 and ncp >= 2 and ncp % 2 == 0
    shift = vecs.bit_length() - 1
    mesh = plsc.VectorSubcoreMesh(core_axis_name="c", subcore_axis_name="s")

    @functools.partial(
        pl.kernel,
        mesh=mesh,
        out_type=jax.ShapeDtypeStruct((nb, seq, d), jnp.float32),
        scratch_types=[
            pltpu.VMEM((nb, ppw), jnp.int32),
            *[pltpu.VMEM((CHUNK, d), jnp.float32) for _ in range(nb + 2)],
            *[pltpu.SemaphoreType.DMA for _ in range(2 * nb + 3)],
        ],
    )
    def emb(x_hbm, table_hbm, pe_hbm, out_hbm, idx_v, *bufs_sems):
        gbuf = list(bufs_sems[:nb])
        pbuf = list(bufs_sems[nb:nb + 2])
        sems = bufs_sems[nb + 2:]
        sg = list(sems[:nb])
        ss = list(sems[nb:2 * nb])
        sp = list(sems[2 * nb:2 * nb + 2])
        si = sems[2 * nb + 2]
        wid = lax.axis_index("s") * NC + lax.axis_index("c")
        pbase = wid * ppw

        def load_pe(c, slot):
            pltpu.async_copy(pe_hbm.at[pl.ds(pbase + c * CHUNK, CHUNK)],
                             pbuf[slot], sp[slot])

        def wait_pe(slot):
            pltpu.make_async_copy(pe_hbm.at[pl.ds(0, CHUNK)],
                                  pbuf[slot], sp[slot]).wait()

        def start(c, b):
            pltpu.async_copy(
                table_hbm.at[idx_v.at[b, pl.ds(c * CHUNK, CHUNK)]],
                gbuf[b], sg[b])

        def wait_gather(b):
            pltpu.make_async_copy(
                table_hbm.at[idx_v.at[b, pl.ds(0, CHUNK)]],
                gbuf[b], sg[b]).wait()

        def wait_scatter(b):
            pltpu.make_async_copy(gbuf[b], out_hbm.at[0, pl.ds(0, CHUNK)],
                                  ss[b]).wait()

        def finish(c, b, pslot):
            wait_gather(b)
            if b == 0:
                wait_pe(pslot)
            g, p = gbuf[b], pbuf[pslot]

            @plsc.parallel_loop(0, CHUNK * vecs, unroll=8)
            def vec(i):
                r = jax.lax.shift_right_logical(i, shift)
                col = (i & (vecs - 1)) * LANES
                plsc.addupdate(g.at[r, pl.ds(col, LANES)],
                               p[r, pl.ds(col, LANES)])

            pltpu.async_copy(g, out_hbm.at[b, pl.ds(pbase + c * CHUNK, CHUNK)],
                             ss[b])

        idx_copy = pltpu.async_copy(
            x_hbm.at[:, pl.ds(pbase, ppw)], idx_v, si)
        load_pe(0, 0)
        load_pe(1, 1)
        idx_copy.wait()

        def body(t, _):
            c0 = 2 * t
            c1 = c0 + 1
            cm = c0 - 1
            nonlast = t < ncp // 2 - 1

            @pl.when(t > 0)
            def _():
                wait_scatter(0)
            start(c0, 0)

            @pl.when(t > 0)
            def _():
                finish(cm, 1, 1)
                wait_scatter(1)
            start(c0, 1)

            @pl.when(t > 0)
            def _():
                finish(cm, 2, 1)
                wait_scatter(2)
            start(c0, 2)

            @pl.when(t > 0)
            def _():
                finish(cm, 3, 1)
                load_pe(c1, 1)
                wait_scatter(3)
            start(c0, 3)
            finish(c0, 0, 0)

            wait_scatter(0)
            start(c1, 0)
            finish(c0, 1, 0)

            wait_scatter(1)
            start(c1, 1)
            finish(c0, 2, 0)

            wait_scatter(2)
            start(c1, 2)
            finish(c0, 3, 0)

            @pl.when(nonlast)
            def _():
                load_pe(c0 + 2, 0)

            wait_scatter(3)
            start(c1, 3)
            finish(c1, 0, 1)
            return 0

        lax.fori_loop(0, ncp // 2, body, 0, unroll=False)

        clast = ncp - 1
        for b in range(1, nb):
            finish(clast, b, 1)
        for b in range(nb):
            wait_scatter(b)

    return emb


def kernel(x, table, pe):
    b, s = x.shape
    vocab, d = table.shape
    return _build(b, s, vocab, d)(x.astype(jnp.int32), table, pe)


# speedup vs baseline: 1.1379x; 1.0889x over previous
import functools

import jax
import jax.numpy as jnp
from jax import lax
from jax.experimental import pallas as pl
from jax.experimental.pallas import tpu as pltpu
from jax.experimental.pallas import tpu_sc as plsc

NC, NS = 2, 16
NW = NC * NS
LANES = 16
CHUNK = 16


@functools.cache
def _build(nb, seq, vocab, d):
    ppw = seq // NW
    ncp = ppw // CHUNK
    vecs = d // LANES
    assert vecs & (vecs - 1) == 0
    assert nb == 4 and ncp >= 2 and ncp % 2 == 0
    shift = vecs.bit_length() - 1
    mesh = plsc.VectorSubcoreMesh(core_axis_name="c", subcore_axis_name="s")

    @functools.partial(
        pl.kernel,
        mesh=mesh,
        out_type=jax.ShapeDtypeStruct((nb, seq, d), jnp.float32),
        scratch_types=[
            pltpu.VMEM((nb, ppw), jnp.int32),
            *[pltpu.VMEM((CHUNK, d), jnp.float32) for _ in range(nb + 2)],
            *[pltpu.SemaphoreType.DMA for _ in range(2 * nb + 3)],
        ],
    )
    def emb(x_hbm, table_hbm, pe_hbm, out_hbm, idx_v, *bufs_sems):
        gbuf = list(bufs_sems[:nb])
        pbuf = list(bufs_sems[nb:nb + 2])
        sems = bufs_sems[nb + 2:]
        sg = list(sems[:nb])
        ss = list(sems[nb:2 * nb])
        sp = list(sems[2 * nb:2 * nb + 2])
        si = sems[2 * nb + 2]
        wid = lax.axis_index("s") * NC + lax.axis_index("c")
        pbase = wid * ppw

        def load_pe(c, slot):
            pltpu.async_copy(pe_hbm.at[pl.ds(pbase + c * CHUNK, CHUNK)],
                             pbuf[slot], sp[slot])

        def wait_pe(slot):
            pltpu.make_async_copy(pe_hbm.at[pl.ds(0, CHUNK)],
                                  pbuf[slot], sp[slot]).wait()

        def start(c, b):
            pltpu.async_copy(
                table_hbm.at[idx_v.at[b, pl.ds(c * CHUNK, CHUNK)]],
                gbuf[b], sg[b])

        def wait_gather(b):
            pltpu.make_async_copy(
                table_hbm.at[idx_v.at[b, pl.ds(0, CHUNK)]],
                gbuf[b], sg[b]).wait()

        def wait_scatter(b):
            pltpu.make_async_copy(gbuf[b], out_hbm.at[0, pl.ds(0, CHUNK)],
                                  ss[b]).wait()

        def finish(c, b, pslot):
            wait_gather(b)
            if b == 0:
                wait_pe(pslot)
            g, p = gbuf[b], pbuf[pslot]

            @plsc.parallel_loop(0, CHUNK * vecs, unroll=8)
            def vec(i):
                r = jax.lax.shift_right_logical(i, shift)
                col = (i & (vecs - 1)) * LANES
                plsc.addupdate(g.at[r, pl.ds(col, LANES)],
                               p[r, pl.ds(col, LANES)])

            pltpu.async_copy(g, out_hbm.at[b, pl.ds(pbase + c * CHUNK, CHUNK)],
                             ss[b])

        idx_copy = pltpu.async_copy(
            x_hbm.at[:, pl.ds(pbase, ppw)], idx_v, si)
        load_pe(0, 0)
        load_pe(1, 1)
        idx_copy.wait()

        def body(t, _):
            c0 = 2 * t
            c1 = c0 + 1
            cm = c0 - 1
            nonlast = t < ncp // 2 - 1

            @pl.when(t > 0)
            def _():
                wait_scatter(0)
            start(c0, 0)

            @pl.when(t > 0)
            def _():
                finish(cm, 2, 1)
                wait_scatter(1)
            start(c0, 1)

            @pl.when(t > 0)
            def _():
                finish(cm, 3, 1)
                load_pe(c1, 1)
                wait_scatter(2)
            start(c0, 2)
            finish(c0, 0, 0)

            @pl.when(t > 0)
            def _():
                wait_scatter(3)
            start(c0, 3)
            finish(c0, 1, 0)

            wait_scatter(0)
            start(c1, 0)
            finish(c0, 2, 0)

            wait_scatter(1)
            start(c1, 1)
            finish(c0, 3, 0)

            @pl.when(nonlast)
            def _():
                load_pe(c0 + 2, 0)

            wait_scatter(2)
            start(c1, 2)
            finish(c1, 0, 1)

            wait_scatter(3)
            start(c1, 3)
            finish(c1, 1, 1)
            return 0

        lax.fori_loop(0, ncp // 2, body, 0, unroll=False)

        clast = ncp - 1
        for b in range(2, nb):
            finish(clast, b, 1)
        for b in range(nb):
            wait_scatter(b)

    return emb


def kernel(x, table, pe):
    b, s = x.shape
    vocab, d = table.shape
    return _build(b, s, vocab, d)(x.astype(jnp.int32), table, pe)
